# baseline (speedup 1.0000x reference)
import functools

import jax
import jax.numpy as jnp
from jax import lax
from jax.experimental import pallas as pl
from jax.experimental.pallas import tpu as pltpu
from jax.experimental.pallas import tpu_sc as plsc

BATCH = 16384
HIST_LEN = 50
EMBEDDING_DIM = 64
TOTAL = BATCH * HIST_LEN
HTOTAL = TOTAL // 2
NUM_EMB = 1000000

_INFO = plsc.get_sparse_core_info()
NUM_CORES = _INFO.num_cores
NUM_SUBCORES = _INFO.num_subcores
NUM_WORKERS = NUM_CORES * NUM_SUBCORES

PER_WORKER = TOTAL // NUM_WORKERS
HPER_WORKER = PER_WORKER // 2
CHUNK = 256
HCHUNK = CHUNK // 2
STEPS = PER_WORKER // CHUNK
NBUF = 4


def _gather_kernel(table_hbm, idxl_hbm, idxr_hbm, out_hbm,
                   idxl_all, idxr_all, rows_l, rows_r, *sems):
    gsems = sems[:NBUF]
    ssems = sems[NBUF:]
    wid = lax.axis_index("s") * NUM_CORES + lax.axis_index("c")
    hbase = wid * HPER_WORKER

    def gather_pair(i, b):
        return (
            pltpu.make_async_copy(
                table_hbm.at[idxl_all.at[pl.ds(i * HCHUNK, HCHUNK)]],
                rows_l.at[b], gsems[b],
            ),
            pltpu.make_async_copy(
                table_hbm.at[idxr_all.at[pl.ds(i * HCHUNK, HCHUNK)]],
                rows_r.at[b], gsems[b],
            ),
        )

    def store_pair(i, b):
        return (
            pltpu.make_async_copy(
                rows_l.at[b],
                out_hbm.at[pl.ds(hbase + i * HCHUNK, HCHUNK), pl.ds(0, 64)],
                ssems[b],
            ),
            pltpu.make_async_copy(
                rows_r.at[b],
                out_hbm.at[pl.ds(hbase + i * HCHUNK, HCHUNK), pl.ds(64, 64)],
                ssems[b],
            ),
        )

    def start_gather(i, b):
        for c in gather_pair(i, b):
            c.start()

    def wait_gather(i, b):
        for c in gather_pair(i, b):
            c.wait()

    def start_store(i, b):
        for c in store_pair(i, b):
            c.start()

    def wait_store(i, b):
        for c in store_pair(i, b):
            c.wait()

    pltpu.sync_copy(idxl_hbm.at[pl.ds(hbase, HPER_WORKER)], idxl_all)
    pltpu.sync_copy(idxr_hbm.at[pl.ds(hbase, HPER_WORKER)], idxr_all)

    for b in range(NBUF):
        start_gather(b, b)

    @pl.loop(0, STEPS, step=NBUF)
    def _outer(g):
        for b in range(NBUF):
            i = g + b
            wait_gather(i, b)
            start_store(i, b)

            @pl.when(i + NBUF < STEPS)
            def _():
                wait_store(i, b)
                start_gather(i + NBUF, b)

    for b in range(NBUF):
        wait_store(STEPS - NBUF + b, b)


@jax.jit
def _gather(table, idxl, idxr):
    mesh = plsc.VectorSubcoreMesh(core_axis_name="c", subcore_axis_name="s")
    run = functools.partial(
        pl.kernel,
        mesh=mesh,
        out_type=jax.ShapeDtypeStruct((HTOTAL, 128), jnp.float32),
        scratch_types=[
            pltpu.VMEM((HPER_WORKER,), jnp.int32),
            pltpu.VMEM((HPER_WORKER,), jnp.int32),
            pltpu.VMEM((NBUF, HCHUNK, EMBEDDING_DIM), jnp.float32),
            pltpu.VMEM((NBUF, HCHUNK, EMBEDDING_DIM), jnp.float32),
        ]
        + [pltpu.SemaphoreType.DMA] * (2 * NBUF),
        compiler_params=pltpu.CompilerParams(use_tc_tiling_on_sc=False),
    )(_gather_kernel)
    return run(table, idxl, idxr)


def kernel(token_ids, embedding_matrix):
    idx_flat = token_ids.reshape(TOTAL).astype(jnp.int32)
    idxl = idx_flat[0::2]
    idxr = idx_flat[1::2]
    out2 = _gather(embedding_matrix, idxl, idxr)
    return out2.reshape(BATCH, HIST_LEN, EMBEDDING_DIM)


# speedup vs baseline: 1.0018x; 1.0018x over previous
import functools

import jax
import jax.numpy as jnp
from jax import lax
from jax.experimental import pallas as pl
from jax.experimental.pallas import tpu as pltpu
from jax.experimental.pallas import tpu_sc as plsc

BATCH = 16384
HIST_LEN = 50
EMBEDDING_DIM = 64
TOTAL = BATCH * HIST_LEN

_INFO = plsc.get_sparse_core_info()
NUM_CORES = _INFO.num_cores
NUM_SUBCORES = _INFO.num_subcores
NUM_WORKERS = NUM_CORES * NUM_SUBCORES

BATCH_PER_WORKER = BATCH // NUM_WORKERS
PER_WORKER = BATCH_PER_WORKER * HIST_LEN
GB = 4
CHUNK = GB * HIST_LEN
STEPS = BATCH_PER_WORKER // GB
NBUF = 4


def _gather_kernel(table_hbm, idx_hbm, out_hbm, idx_all, rows, *sems):
    gsems = sems[:NBUF]
    ssems = sems[NBUF:]
    wid = lax.axis_index("s") * NUM_CORES + lax.axis_index("c")
    row_base = wid * PER_WORKER
    batch_base = wid * BATCH_PER_WORKER

    def idx_slice(i):
        return idx_all.at[pl.ds(i * CHUNK, CHUNK)]

    def start_gather(i, b):
        pltpu.async_copy(table_hbm.at[idx_slice(i)], rows.at[b], gsems[b])

    def wait_gather(i, b):
        pltpu.make_async_copy(
            table_hbm.at[idx_slice(i)], rows.at[b], gsems[b]
        ).wait()

    def start_store(i, b):
        for g in range(GB):
            pltpu.async_copy(
                rows.at[b, pl.ds(g * HIST_LEN, HIST_LEN)],
                out_hbm.at[batch_base + i * GB + g],
                ssems[b],
            )

    def wait_store(i, b):
        for g in range(GB):
            pltpu.make_async_copy(
                rows.at[b, pl.ds(g * HIST_LEN, HIST_LEN)],
                out_hbm.at[batch_base + i * GB + g],
                ssems[b],
            ).wait()

    pltpu.sync_copy(idx_hbm.at[pl.ds(row_base, PER_WORKER)], idx_all)

    for b in range(NBUF):
        start_gather(b, b)

    @pl.loop(0, STEPS, step=NBUF)
    def _outer(g):
        for b in range(NBUF):
            i = g + b
            wait_gather(i, b)
            start_store(i, b)

            @pl.when(i + NBUF < STEPS)
            def _():
                wait_store(i, b)
                start_gather(i + NBUF, b)

    for b in range(NBUF):
        wait_store(STEPS - NBUF + b, b)


@jax.jit
def _gather(table, idx_flat):
    mesh = plsc.VectorSubcoreMesh(core_axis_name="c", subcore_axis_name="s")
    run = functools.partial(
        pl.kernel,
        mesh=mesh,
        out_type=jax.ShapeDtypeStruct(
            (BATCH, HIST_LEN, EMBEDDING_DIM), jnp.float32
        ),
        scratch_types=[
            pltpu.VMEM((PER_WORKER,), jnp.int32),
            pltpu.VMEM((NBUF, CHUNK, EMBEDDING_DIM), jnp.float32),
        ]
        + [pltpu.SemaphoreType.DMA] * (2 * NBUF),
        compiler_params=pltpu.CompilerParams(use_tc_tiling_on_sc=False),
    )(_gather_kernel)
    return run(table, idx_flat)


def kernel(token_ids, embedding_matrix):
    idx_flat = token_ids.reshape(TOTAL).astype(jnp.int32)
    return _gather(embedding_matrix, idx_flat)


# speedup vs baseline: 1.0037x; 1.0020x over previous
import functools

import jax
import jax.numpy as jnp
from jax import lax
from jax.experimental import pallas as pl
from jax.experimental.pallas import tpu as pltpu
from jax.experimental.pallas import tpu_sc as plsc

BATCH = 16384
HIST_LEN = 50
EMBEDDING_DIM = 64
TOTAL = BATCH * HIST_LEN

_INFO = plsc.get_sparse_core_info()
NUM_CORES = _INFO.num_cores
NUM_SUBCORES = _INFO.num_subcores
NUM_WORKERS = NUM_CORES * NUM_SUBCORES

BATCH_PER_WORKER = BATCH // NUM_WORKERS
PER_WORKER = BATCH_PER_WORKER * HIST_LEN
GB = 8
CHUNK = GB * HIST_LEN
STEPS = BATCH_PER_WORKER // GB
NBUF = 2


def _gather_kernel(table_hbm, idx_hbm, out_hbm, idx_all, rows, *sems):
    gsems = sems[:NBUF]
    ssems = sems[NBUF:]
    wid = lax.axis_index("s") * NUM_CORES + lax.axis_index("c")
    row_base = wid * PER_WORKER
    batch_base = wid * BATCH_PER_WORKER

    def idx_slice(i):
        return idx_all.at[pl.ds(i * CHUNK, CHUNK)]

    def start_gather(i, b):
        pltpu.async_copy(table_hbm.at[idx_slice(i)], rows.at[b], gsems[b])

    def wait_gather(i, b):
        pltpu.make_async_copy(
            table_hbm.at[idx_slice(i)], rows.at[b], gsems[b]
        ).wait()

    def start_store(i, b):
        for g in range(GB):
            pltpu.async_copy(
                rows.at[b, pl.ds(g * HIST_LEN, HIST_LEN)],
                out_hbm.at[batch_base + i * GB + g],
                ssems[b],
            )

    def wait_store(i, b):
        for g in range(GB):
            pltpu.make_async_copy(
                rows.at[b, pl.ds(g * HIST_LEN, HIST_LEN)],
                out_hbm.at[batch_base + i * GB + g],
                ssems[b],
            ).wait()

    pltpu.sync_copy(idx_hbm.at[pl.ds(row_base, PER_WORKER)], idx_all)

    for b in range(NBUF):
        start_gather(b, b)

    @pl.loop(0, STEPS, step=NBUF)
    def _outer(g):
        for b in range(NBUF):
            i = g + b
            wait_gather(i, b)
            start_store(i, b)

            @pl.when(i + NBUF < STEPS)
            def _():
                wait_store(i, b)
                start_gather(i + NBUF, b)

    for b in range(NBUF):
        wait_store(STEPS - NBUF + b, b)


@jax.jit
def _gather(table, idx_flat):
    mesh = plsc.VectorSubcoreMesh(core_axis_name="c", subcore_axis_name="s")
    run = functools.partial(
        pl.kernel,
        mesh=mesh,
        out_type=jax.ShapeDtypeStruct(
            (BATCH, HIST_LEN, EMBEDDING_DIM), jnp.float32
        ),
        scratch_types=[
            pltpu.VMEM((PER_WORKER,), jnp.int32),
            pltpu.VMEM((NBUF, CHUNK, EMBEDDING_DIM), jnp.float32),
        ]
        + [pltpu.SemaphoreType.DMA] * (2 * NBUF),
        compiler_params=pltpu.CompilerParams(use_tc_tiling_on_sc=False),
    )(_gather_kernel)
    return run(table, idx_flat)


def kernel(token_ids, embedding_matrix):
    idx_flat = token_ids.reshape(TOTAL).astype(jnp.int32)
    return _gather(embedding_matrix, idx_flat)
